# baseline (speedup 1.0000x reference)
import functools

import jax
import jax.numpy as jnp
import numpy as np
from jax import lax
from jax.experimental import pallas as pl
from jax.experimental.pallas import tpu as pltpu
from jax.experimental.pallas import tpu_sc as plsc

T = 32768
E = 64
K = 8
ONE_M_EPS2 = np.float32(0.98)
INV_ONE_M_EPS2 = np.float32(1.0 / 0.98)

NUM_CORES = 2
NUM_SUBCORES = 16
NW = NUM_CORES * NUM_SUBCORES
ROWS_PER_W = T // NW
GROUPS = ROWS_PER_W // 16
U = 16
NACC = 4

_NEG_INF = np.float32(-np.inf)
_ZERO = np.float32(0.0)
_ONE = np.float32(1.0)


def _loop(n, body, init):
  if n == 1:
    return body(0, init)
  return lax.fori_loop(0, n, body, init)


def _merge_max(va, ia, vb, ib):
  take_b = (vb > va) | ((vb == va) & (ib < ia))
  return jnp.where(take_b, vb, va), jnp.where(take_b, ib, ia)


def _router_body(logits_hbm, out_idx_hbm, out_val_hbm, l_v, oi_v, ov_v):
  wid = lax.axis_index("s") * NUM_CORES + lax.axis_index("c")
  rbase = wid * ROWS_PER_W
  lanes = jnp.arange(16, dtype=jnp.int32)

  pltpu.sync_copy(logits_hbm.at[:, pl.ds(rbase, ROWS_PER_W)], l_v)

  def group_body(g):
    g16 = g * 16
    rloc = g16 + lanes

    def init_body(t, carry):
      mvs, mis = carry
      mvs, mis = list(mvs), list(mis)
      e0 = t * U
      for u in range(U):
        a = u % NACC
        v = l_v[e0 + u, pl.ds(g16, 16)]
        gt = v > mvs[a]
        mvs[a] = jnp.where(gt, v, mvs[a])
        mis[a] = jnp.where(gt, jnp.full((16,), e0 + u, jnp.int32), mis[a])
      return tuple(mvs), tuple(mis)

    ninf16 = jnp.full((16,), _NEG_INF)
    zero16i = jnp.zeros((16,), jnp.int32)
    zero16f = jnp.zeros((16,), jnp.float32)
    mvs, mis = _loop(
        E // U, init_body,
        ((ninf16,) * NACC, (zero16i,) * NACC))
    mv, mi = mvs[0], mis[0]
    for a in range(1, NACC):
      mv, mi = _merge_max(mv, mi, mvs[a], mis[a])

    def topk_step(k, carry):
      mv, mi = carry
      plsc.store_scatter(l_v, [mi, rloc], ninf16)
      cut = jnp.where(mv >= _ZERO, ONE_M_EPS2 * mv, INV_ONE_M_EPS2 * mv)

      def fused_body(t, carry):
        dens, nmvs, nmis = carry
        dens, nmvs, nmis = list(dens), list(nmvs), list(nmis)
        e0 = t * U
        for u in range(U):
          a = u % NACC
          v = l_v[e0 + u, pl.ds(g16, 16)]
          ev = jnp.exp(v - mv)
          dens[a] = dens[a] + jnp.where(v >= cut, ev, _ZERO)
          gt = v > nmvs[a]
          nmvs[a] = jnp.where(gt, v, nmvs[a])
          nmis[a] = jnp.where(gt, jnp.full((16,), e0 + u, jnp.int32),
                              nmis[a])
        return tuple(dens), tuple(nmvs), tuple(nmis)

      dens, nmvs, nmis = _loop(
          E // U, fused_body,
          ((zero16f,) * NACC, (ninf16,) * NACC, (zero16i,) * NACC))
      den = (dens[0] + dens[1]) + (dens[2] + dens[3])
      nmv, nmi = nmvs[0], nmis[0]
      for a in range(1, NACC):
        nmv, nmi = _merge_max(nmv, nmi, nmvs[a], nmis[a])

      scale = _ONE / (den + _ONE)
      kvec = jnp.full((16,), k, jnp.int32)
      plsc.store_scatter(oi_v, [kvec, rloc], mi)
      plsc.store_scatter(ov_v, [kvec, rloc], scale)
      return nmv, nmi

    lax.fori_loop(0, K, topk_step, (mv, mi))

  plsc.parallel_loop(0, GROUPS, unroll=2)(group_body)
  pltpu.sync_copy(oi_v, out_idx_hbm.at[:, pl.ds(rbase, ROWS_PER_W)])
  pltpu.sync_copy(ov_v, out_val_hbm.at[:, pl.ds(rbase, ROWS_PER_W)])


@jax.jit
def _router(router_logits):
  mesh = plsc.VectorSubcoreMesh(
      core_axis_name="c", subcore_axis_name="s", num_cores=NUM_CORES)
  f = functools.partial(
      pl.kernel,
      mesh=mesh,
      compiler_params=pltpu.CompilerParams(needs_layout_passes=False),
      out_type=[
          jax.ShapeDtypeStruct((K, T), jnp.int32),
          jax.ShapeDtypeStruct((K, T), jnp.float32),
      ],
      scratch_types=[
          pltpu.VMEM((E, ROWS_PER_W), jnp.float32),
          pltpu.VMEM((K, ROWS_PER_W), jnp.int32),
          pltpu.VMEM((K, ROWS_PER_W), jnp.float32),
      ],
  )(_router_body)
  oi, ov = f(router_logits.T)
  return oi.T, ov.T


def kernel(router_logits):
  return _router(router_logits)


# speedup vs baseline: 1.0021x; 1.0021x over previous
import functools

import jax
import jax.numpy as jnp
import numpy as np
from jax import lax
from jax.experimental import pallas as pl
from jax.experimental.pallas import tpu as pltpu
from jax.experimental.pallas import tpu_sc as plsc

T = 32768
E = 64
K = 8
ONE_M_EPS2 = np.float32(0.98)
INV_ONE_M_EPS2 = np.float32(1.0 / 0.98)

NUM_CORES = 2
NUM_SUBCORES = 16
NW = NUM_CORES * NUM_SUBCORES
ROWS_PER_W = T // NW
GROUPS = ROWS_PER_W // 16
U = 16
NACC = 4

_NEG_INF = np.float32(-np.inf)
_ZERO = np.float32(0.0)
_ONE = np.float32(1.0)


def _loop(n, body, init):
  if n == 1:
    return body(0, init)
  return lax.fori_loop(0, n, body, init)


def _merge_max(va, ia, vb, ib):
  take_b = (vb > va) | ((vb == va) & (ib < ia))
  return jnp.where(take_b, vb, va), jnp.where(take_b, ib, ia)


def _router_body(logits_hbm, out_idx_hbm, out_val_hbm, l_v, oi_v, ov_v):
  wid = lax.axis_index("s") * NUM_CORES + lax.axis_index("c")
  rbase = wid * ROWS_PER_W
  lanes = jnp.arange(16, dtype=jnp.int32)

  pltpu.sync_copy(logits_hbm.at[:, pl.ds(rbase, ROWS_PER_W)], l_v)

  def group_body(g):
    g16 = g * 16
    rloc = g16 + lanes

    def init_body(t, carry):
      mvs, mis = carry
      mvs, mis = list(mvs), list(mis)
      e0 = t * U
      for u in range(U):
        a = u % NACC
        v = l_v[e0 + u, pl.ds(g16, 16)]
        gt = v > mvs[a]
        mvs[a] = jnp.where(gt, v, mvs[a])
        mis[a] = jnp.where(gt, jnp.full((16,), e0 + u, jnp.int32), mis[a])
      return tuple(mvs), tuple(mis)

    ninf16 = jnp.full((16,), _NEG_INF)
    zero16i = jnp.zeros((16,), jnp.int32)
    zero16f = jnp.zeros((16,), jnp.float32)
    mvs, mis = _loop(
        E // U, init_body,
        ((ninf16,) * NACC, (zero16i,) * NACC))
    mv, mi = mvs[0], mis[0]
    for a in range(1, NACC):
      mv, mi = _merge_max(mv, mi, mvs[a], mis[a])

    def topk_step(k, carry):
      mv, mi = carry
      plsc.store_scatter(l_v, [mi, rloc], ninf16)
      cut = jnp.where(mv >= _ZERO, ONE_M_EPS2 * mv, INV_ONE_M_EPS2 * mv)

      def fused_body(t, carry):
        dens, nmvs, nmis = carry
        dens, nmvs, nmis = list(dens), list(nmvs), list(nmis)
        e0 = t * U
        for u in range(U):
          a = u % NACC
          v = l_v[e0 + u, pl.ds(g16, 16)]
          ev = jnp.exp(v - mv)
          dens[a] = dens[a] + jnp.where(v >= cut, ev, _ZERO)
          gt = v > nmvs[a]
          nmvs[a] = jnp.where(gt, v, nmvs[a])
          nmis[a] = jnp.where(gt, jnp.full((16,), e0 + u, jnp.int32),
                              nmis[a])
        return tuple(dens), tuple(nmvs), tuple(nmis)

      dens, nmvs, nmis = _loop(
          E // U, fused_body,
          ((zero16f,) * NACC, (ninf16,) * NACC, (zero16i,) * NACC))
      den = (dens[0] + dens[1]) + (dens[2] + dens[3])
      nmv, nmi = nmvs[0], nmis[0]
      for a in range(1, NACC):
        nmv, nmi = _merge_max(nmv, nmi, nmvs[a], nmis[a])

      scale = _ONE / (den + _ONE)
      kvec = jnp.full((16,), k, jnp.int32)
      plsc.store_scatter(oi_v, [kvec, rloc], mi)
      plsc.store_scatter(ov_v, [kvec, rloc], scale)
      return nmv, nmi

    lax.fori_loop(0, K, topk_step, (mv, mi))

  plsc.parallel_loop(0, GROUPS)(group_body)
  pltpu.sync_copy(oi_v, out_idx_hbm.at[:, pl.ds(rbase, ROWS_PER_W)])
  pltpu.sync_copy(ov_v, out_val_hbm.at[:, pl.ds(rbase, ROWS_PER_W)])


@jax.jit
def _router(router_logits):
  mesh = plsc.VectorSubcoreMesh(
      core_axis_name="c", subcore_axis_name="s", num_cores=NUM_CORES)
  f = functools.partial(
      pl.kernel,
      mesh=mesh,
      compiler_params=pltpu.CompilerParams(needs_layout_passes=False),
      out_type=[
          jax.ShapeDtypeStruct((K, T), jnp.int32),
          jax.ShapeDtypeStruct((K, T), jnp.float32),
      ],
      scratch_types=[
          pltpu.VMEM((E, ROWS_PER_W), jnp.float32),
          pltpu.VMEM((K, ROWS_PER_W), jnp.int32),
          pltpu.VMEM((K, ROWS_PER_W), jnp.float32),
      ],
  )(_router_body)
  oi, ov = f(router_logits.T)
  return oi.T, ov.T


def kernel(router_logits):
  return _router(router_logits)
